# baseline (speedup 1.0000x reference)
import functools

import numpy as np
import jax
import jax.numpy as jnp
from jax import lax
from jax.experimental import pallas as pl
from jax.experimental.pallas import tpu as pltpu
from jax.experimental.pallas import tpu_sc as plsc

_B = 128
_V = 100000
_TILE = 2048
_GRID = (_V + _TILE - 1) // _TILE

_K1 = np.uint32(0)
_K2 = np.uint32(42)
_KS2 = np.uint32(0x1BD11BDA) ^ _K1 ^ _K2
_R0 = (13, 15, 26, 6)
_R1 = (17, 29, 16, 24)
_TINY = np.float32(np.finfo(np.float32).tiny)
_NEG_INF = np.float32(-np.inf)
_BIG = np.int32(np.iinfo(np.int32).max)


def _rotl(x, r):
    return (x << np.uint32(r)) | (x >> np.uint32(32 - r))


def _rounds(x0, x1, rots):
    for r in rots:
        x0 = x0 + x1
        x1 = _rotl(x1, r) ^ x0
    return x0, x1


def _threefry_bits(x1):
    x0 = x1
    x1 = _rotl(x1, _R0[0]) ^ x0
    x0, x1 = _rounds(x0, x1, _R0[1:])
    x0, x1 = x0 + _K2, x1 + (_KS2 + np.uint32(1))
    x0, x1 = _rounds(x0, x1, _R1)
    x0, x1 = x0 + _KS2, x1 + (_K1 + np.uint32(2))
    x0, x1 = _rounds(x0, x1, _R0)
    x0, x1 = x0 + _K1, x1 + (_K2 + np.uint32(3))
    x0, x1 = _rounds(x0, x1, _R1)
    x0, x1 = x0 + _K2, x1 + (_KS2 + np.uint32(4))
    x0, x1 = _rounds(x0, x1, _R0)
    return (x0 + _KS2) ^ (x1 + (_K1 + np.uint32(5)))


def _sample_kernel(lp_ref, out_ref, best_val, best_idx):
    j = pl.program_id(0)
    shape = (_B, _TILE)

    b = jax.lax.broadcasted_iota(jnp.uint32, shape, 0)
    vglob = jax.lax.broadcasted_iota(jnp.int32, shape, 1) + j * _TILE

    bits = _threefry_bits((b * np.uint32(_V) + _K2) + vglob.astype(jnp.uint32))

    fb = (bits >> np.uint32(9)) | np.uint32(0x3F800000)
    u = jnp.maximum(
        jax.lax.bitcast_convert_type(fb, jnp.float32) - np.float32(1.0), _TINY)
    g = -jnp.log(-jnp.log(u))

    t = g + jnp.transpose(lp_ref[...])
    t = jnp.where(vglob < _V, t, _NEG_INF)

    m = jnp.max(t, axis=1, keepdims=True)
    a = jnp.min(jnp.where(t == m, vglob, _BIG), axis=1, keepdims=True)

    @pl.when(j == 0)
    def _():
        best_val[...] = m
        best_idx[...] = a

    @pl.when(j != 0)
    def _():
        upd = m > best_val[...]
        best_val[...] = jnp.where(upd, m, best_val[...])
        best_idx[...] = jnp.where(upd, a, best_idx[...])

    @pl.when(j == _GRID - 1)
    def _():
        out_ref[...] = best_idx[...]


_V0_SC = 81920
_WSC = 16384
_NW = 32
_ROWS_W = _WSC // _NW


def _sc_bits_body(out_hbm, buf, sem):
    wid = lax.axis_index("c") * 16 + lax.axis_index("s")
    row0 = wid * _ROWS_W
    lane_v = jax.lax.broadcasted_iota(jnp.uint32, (16,), 0) * np.uint32(_V)

    def row_body(r, _):
        v = row0 + r
        base = (np.uint32(_V0_SC) + _K2) + v.astype(jnp.uint32)
        for c in range(8):
            x1 = (lane_v + np.uint32(16 * c * _V)) + base
            buf[r, pl.ds(16 * c, 16)] = _threefry_bits(x1)
        return _

    lax.fori_loop(0, _ROWS_W, row_body, 0)
    pltpu.async_copy(buf, out_hbm.at[pl.ds(row0, _ROWS_W), :], sem).wait()


def _sc_bits():
    fn = pl.kernel(
        _sc_bits_body,
        out_type=jax.ShapeDtypeStruct((_WSC, _B), jnp.uint32),
        mesh=plsc.VectorSubcoreMesh(core_axis_name="c", subcore_axis_name="s"),
        scratch_types=[
            pltpu.VMEM((_ROWS_W, _B), jnp.uint32),
            pltpu.SemaphoreType.DMA,
        ],
    )
    return fn()


def kernel(log_p):
    out = pl.pallas_call(
        _sample_kernel,
        grid=(_GRID,),
        in_specs=[pl.BlockSpec((_TILE, _B), lambda j: (j, 0))],
        out_specs=pl.BlockSpec((_B, 1), lambda j: (0, 0)),
        out_shape=jax.ShapeDtypeStruct((_B, 1), jnp.int32),
        scratch_shapes=[
            pltpu.VMEM((_B, 1), jnp.float32),
            pltpu.VMEM((_B, 1), jnp.int32),
        ],
        compiler_params=pltpu.CompilerParams(
            dimension_semantics=("arbitrary",)),
    )(log_p.T)
    bits = _sc_bits()
    return out.reshape(_B) ^ bits[0, :].astype(jnp.int32)


# speedup vs baseline: 1.1911x; 1.1911x over previous
import functools

import numpy as np
import jax
import jax.numpy as jnp
from jax import lax
from jax.experimental import pallas as pl
from jax.experimental.pallas import tpu as pltpu
from jax.experimental.pallas import tpu_sc as plsc

_B = 128
_V = 100000
_TILE = 2048

_V0_SC = 73728
_WSC = 24576
_NW = 32
_ROWS_W = _WSC // _NW
_GRID1 = _V0_SC // _TILE
_GRID2 = 13

_K1 = np.uint32(0)
_K2 = np.uint32(42)
_KS2 = np.uint32(0x1BD11BDA) ^ _K1 ^ _K2
_R0 = (13, 15, 26, 6)
_R1 = (17, 29, 16, 24)
_TINY = np.float32(np.finfo(np.float32).tiny)
_NEG_INF = np.float32(-np.inf)
_BIG = np.int32(np.iinfo(np.int32).max)


def _rotl(x, r):
    return (x << np.uint32(r)) | (x >> np.uint32(32 - r))


def _rounds(x0, x1, rots):
    for r in rots:
        x0 = x0 + x1
        x1 = _rotl(x1, r) ^ x0
    return x0, x1


def _threefry_bits(x1):
    x0 = x1
    x1 = _rotl(x1, _R0[0]) ^ x0
    x0, x1 = _rounds(x0, x1, _R0[1:])
    x0, x1 = x0 + _K2, x1 + (_KS2 + np.uint32(1))
    x0, x1 = _rounds(x0, x1, _R1)
    x0, x1 = x0 + _KS2, x1 + (_K1 + np.uint32(2))
    x0, x1 = _rounds(x0, x1, _R0)
    x0, x1 = x0 + _K1, x1 + (_K2 + np.uint32(3))
    x0, x1 = _rounds(x0, x1, _R1)
    x0, x1 = x0 + _K2, x1 + (_KS2 + np.uint32(4))
    x0, x1 = _rounds(x0, x1, _R0)
    return (x0 + _KS2) ^ (x1 + (_K1 + np.uint32(5)))


def _gumbel_from_bits(bits):
    fb = (bits >> np.uint32(9)) | np.uint32(0x3F800000)
    u = jnp.maximum(
        jax.lax.bitcast_convert_type(fb, jnp.float32) - np.float32(1.0), _TINY)
    return -jnp.log(-jnp.log(u))


def _tile_best(t, vglob):
    m = jnp.max(t, axis=1, keepdims=True)
    a = jnp.min(jnp.where(t == m, vglob, _BIG), axis=1, keepdims=True)
    return m, a


def _race_update(bv_ref, bi_ref, m, a):
    upd = m > bv_ref[...]
    bv_ref[...] = jnp.where(upd, m, bv_ref[...])
    bi_ref[...] = jnp.where(upd, a, bi_ref[...])


def _main_kernel(lp_ref, bv_ref, bi_ref):
    j = pl.program_id(0)
    shape = (_B, _TILE)

    b = jax.lax.broadcasted_iota(jnp.uint32, shape, 0)
    vglob = jax.lax.broadcasted_iota(jnp.int32, shape, 1) + j * _TILE

    bits = _threefry_bits((b * np.uint32(_V) + _K2) + vglob.astype(jnp.uint32))
    t = _gumbel_from_bits(bits) + jnp.transpose(lp_ref[...])
    m, a = _tile_best(t, vglob)

    @pl.when(j == 0)
    def _():
        bv_ref[...] = m
        bi_ref[...] = a

    @pl.when(j != 0)
    def _():
        _race_update(bv_ref, bi_ref, m, a)


def _finish_kernel(lp_ref, bits_ref, bv_in, bi_in, bv_ref, bi_ref):
    j = pl.program_id(0)
    shape = (_B, _TILE)
    vglob = jax.lax.broadcasted_iota(jnp.int32, shape, 1) + (j * _TILE + _V0_SC)

    @pl.when(j == 0)
    def _():
        bv_ref[...] = bv_in[...]
        bi_ref[...] = bi_in[...]

    @pl.when(j < _GRID2 - 1)
    def _():
        t = _gumbel_from_bits(jnp.transpose(bits_ref[...])) \
            + jnp.transpose(lp_ref[...])
        m, a = _tile_best(t, vglob)
        _race_update(bv_ref, bi_ref, m, a)

    @pl.when(j == _GRID2 - 1)
    def _():
        b = jax.lax.broadcasted_iota(jnp.uint32, shape, 0)
        bits = _threefry_bits(
            (b * np.uint32(_V) + _K2) + vglob.astype(jnp.uint32))
        t = _gumbel_from_bits(bits) + jnp.transpose(lp_ref[...])
        t = jnp.where(vglob < _V, t, _NEG_INF)
        m, a = _tile_best(t, vglob)
        _race_update(bv_ref, bi_ref, m, a)


def _sc_bits_body(out_hbm, buf, sem):
    wid = lax.axis_index("c") * 16 + lax.axis_index("s")
    row0 = wid * _ROWS_W
    lane_v = jax.lax.broadcasted_iota(jnp.uint32, (16,), 0) * np.uint32(_V)

    def row_body(r, carry):
        v = row0 + r
        base = (np.uint32(_V0_SC) + _K2) + v.astype(jnp.uint32)
        for c in range(8):
            x1 = (lane_v + np.uint32(16 * c * _V)) + base
            buf[r, pl.ds(16 * c, 16)] = _threefry_bits(x1)
        return carry

    lax.fori_loop(0, _ROWS_W, row_body, 0)
    pltpu.async_copy(buf, out_hbm.at[pl.ds(row0, _ROWS_W), :], sem).wait()


def _sc_bits():
    fn = pl.kernel(
        _sc_bits_body,
        out_type=jax.ShapeDtypeStruct((_WSC, _B), jnp.uint32),
        mesh=plsc.VectorSubcoreMesh(core_axis_name="c", subcore_axis_name="s"),
        scratch_types=[
            pltpu.VMEM((_ROWS_W, _B), jnp.uint32),
            pltpu.SemaphoreType.DMA,
        ],
    )
    return fn()


_STATE_SPEC = pl.BlockSpec((_B, 1), lambda j: (0, 0))
_STATE_SHAPE = [
    jax.ShapeDtypeStruct((_B, 1), jnp.float32),
    jax.ShapeDtypeStruct((_B, 1), jnp.int32),
]


def kernel(log_p):
    lp_t = log_p.T
    bits = _sc_bits()

    bv, bi = pl.pallas_call(
        _main_kernel,
        grid=(_GRID1,),
        in_specs=[pl.BlockSpec((_TILE, _B), lambda j: (j, 0))],
        out_specs=[_STATE_SPEC, _STATE_SPEC],
        out_shape=_STATE_SHAPE,
        compiler_params=pltpu.CompilerParams(
            dimension_semantics=("arbitrary",)),
    )(lp_t)

    _, bi2 = pl.pallas_call(
        _finish_kernel,
        grid=(_GRID2,),
        in_specs=[
            pl.BlockSpec((_TILE, _B), lambda j: (j + _GRID1, 0)),
            pl.BlockSpec((_TILE, _B), lambda j: (jnp.minimum(j, _GRID2 - 2), 0)),
            _STATE_SPEC,
            _STATE_SPEC,
        ],
        out_specs=[_STATE_SPEC, _STATE_SPEC],
        out_shape=_STATE_SHAPE,
        compiler_params=pltpu.CompilerParams(
            dimension_semantics=("arbitrary",)),
    )(lp_t, bits, bv, bi)

    return bi2.reshape(_B)


# speedup vs baseline: 1.2096x; 1.0155x over previous
import functools

import numpy as np
import jax
import jax.numpy as jnp
from jax import lax
from jax.experimental import pallas as pl
from jax.experimental.pallas import tpu as pltpu
from jax.experimental.pallas import tpu_sc as plsc

_B = 128
_V = 100000
_TILE = 2048

_V0_SC = 71680
_WSC = 26624
_NW = 32
_ROWS_W = _WSC // _NW
_GRID1 = _V0_SC // _TILE
_GRID2 = 14

_K1 = np.uint32(0)
_K2 = np.uint32(42)
_KS2 = np.uint32(0x1BD11BDA) ^ _K1 ^ _K2
_R0 = (13, 15, 26, 6)
_R1 = (17, 29, 16, 24)
_TINY = np.float32(np.finfo(np.float32).tiny)
_NEG_INF = np.float32(-np.inf)
_BIG = np.int32(np.iinfo(np.int32).max)


def _rotl(x, r):
    return (x << np.uint32(r)) | (x >> np.uint32(32 - r))


def _rounds(x0, x1, rots):
    for r in rots:
        x0 = x0 + x1
        x1 = _rotl(x1, r) ^ x0
    return x0, x1


def _threefry_bits(x1):
    x0 = x1
    x1 = _rotl(x1, _R0[0]) ^ x0
    x0, x1 = _rounds(x0, x1, _R0[1:])
    x0, x1 = x0 + _K2, x1 + (_KS2 + np.uint32(1))
    x0, x1 = _rounds(x0, x1, _R1)
    x0, x1 = x0 + _KS2, x1 + (_K1 + np.uint32(2))
    x0, x1 = _rounds(x0, x1, _R0)
    x0, x1 = x0 + _K1, x1 + (_K2 + np.uint32(3))
    x0, x1 = _rounds(x0, x1, _R1)
    x0, x1 = x0 + _K2, x1 + (_KS2 + np.uint32(4))
    x0, x1 = _rounds(x0, x1, _R0)
    return (x0 + _KS2) ^ (x1 + (_K1 + np.uint32(5)))


def _gumbel_from_bits(bits):
    fb = (bits >> np.uint32(9)) | np.uint32(0x3F800000)
    u = jnp.maximum(
        jax.lax.bitcast_convert_type(fb, jnp.float32) - np.float32(1.0), _TINY)
    return -jnp.log(-jnp.log(u))


def _tile_best(t, vglob):
    m = jnp.max(t, axis=1, keepdims=True)
    a = jnp.min(jnp.where(t == m, vglob, _BIG), axis=1, keepdims=True)
    return m, a


def _race_update(bv_ref, bi_ref, m, a):
    upd = m > bv_ref[...]
    bv_ref[...] = jnp.where(upd, m, bv_ref[...])
    bi_ref[...] = jnp.where(upd, a, bi_ref[...])


def _main_kernel(lp_ref, bv_ref, bi_ref):
    j = pl.program_id(0)
    shape = (_B, _TILE)

    b = jax.lax.broadcasted_iota(jnp.uint32, shape, 0)
    vglob = jax.lax.broadcasted_iota(jnp.int32, shape, 1) + j * _TILE

    bits = _threefry_bits((b * np.uint32(_V) + _K2) + vglob.astype(jnp.uint32))
    t = _gumbel_from_bits(bits) + jnp.transpose(lp_ref[...])
    m, a = _tile_best(t, vglob)

    @pl.when(j == 0)
    def _():
        bv_ref[...] = m
        bi_ref[...] = a

    @pl.when(j != 0)
    def _():
        _race_update(bv_ref, bi_ref, m, a)


def _finish_kernel(lp_ref, bits_ref, bv_in, bi_in, bv_ref, bi_ref):
    j = pl.program_id(0)
    shape = (_B, _TILE)
    vglob = jax.lax.broadcasted_iota(jnp.int32, shape, 1) + (j * _TILE + _V0_SC)

    @pl.when(j == 0)
    def _():
        bv_ref[...] = bv_in[...]
        bi_ref[...] = bi_in[...]

    @pl.when(j < _GRID2 - 1)
    def _():
        t = _gumbel_from_bits(jnp.transpose(bits_ref[...])) \
            + jnp.transpose(lp_ref[...])
        m, a = _tile_best(t, vglob)
        _race_update(bv_ref, bi_ref, m, a)

    @pl.when(j == _GRID2 - 1)
    def _():
        b = jax.lax.broadcasted_iota(jnp.uint32, shape, 0)
        bits = _threefry_bits(
            (b * np.uint32(_V) + _K2) + vglob.astype(jnp.uint32))
        t = _gumbel_from_bits(bits) + jnp.transpose(lp_ref[...])
        t = jnp.where(vglob < _V, t, _NEG_INF)
        m, a = _tile_best(t, vglob)
        _race_update(bv_ref, bi_ref, m, a)


def _sc_bits_body(out_hbm, buf, sem):
    wid = lax.axis_index("c") * 16 + lax.axis_index("s")
    row0 = wid * _ROWS_W
    lane_v = jax.lax.broadcasted_iota(jnp.uint32, (16,), 0) * np.uint32(_V)

    def row_body(r, carry):
        v = row0 + r
        base = (np.uint32(_V0_SC) + _K2) + v.astype(jnp.uint32)
        for c in range(8):
            x1 = (lane_v + np.uint32(16 * c * _V)) + base
            buf[r, pl.ds(16 * c, 16)] = _threefry_bits(x1)
        return carry

    lax.fori_loop(0, _ROWS_W, row_body, 0)
    pltpu.async_copy(buf, out_hbm.at[pl.ds(row0, _ROWS_W), :], sem).wait()


def _sc_bits():
    fn = pl.kernel(
        _sc_bits_body,
        out_type=jax.ShapeDtypeStruct((_WSC, _B), jnp.uint32),
        mesh=plsc.VectorSubcoreMesh(core_axis_name="c", subcore_axis_name="s"),
        scratch_types=[
            pltpu.VMEM((_ROWS_W, _B), jnp.uint32),
            pltpu.SemaphoreType.DMA,
        ],
    )
    return fn()


_STATE_SPEC = pl.BlockSpec((_B, 1), lambda j: (0, 0))
_STATE_SHAPE = [
    jax.ShapeDtypeStruct((_B, 1), jnp.float32),
    jax.ShapeDtypeStruct((_B, 1), jnp.int32),
]


def kernel(log_p):
    lp_t = log_p.T
    bits = _sc_bits()

    bv, bi = pl.pallas_call(
        _main_kernel,
        grid=(_GRID1,),
        in_specs=[pl.BlockSpec((_TILE, _B), lambda j: (j, 0))],
        out_specs=[_STATE_SPEC, _STATE_SPEC],
        out_shape=_STATE_SHAPE,
        compiler_params=pltpu.CompilerParams(
            dimension_semantics=("arbitrary",)),
    )(lp_t)

    _, bi2 = pl.pallas_call(
        _finish_kernel,
        grid=(_GRID2,),
        in_specs=[
            pl.BlockSpec((_TILE, _B), lambda j: (j + _GRID1, 0)),
            pl.BlockSpec((_TILE, _B), lambda j: (jnp.minimum(j, _GRID2 - 2), 0)),
            _STATE_SPEC,
            _STATE_SPEC,
        ],
        out_specs=[_STATE_SPEC, _STATE_SPEC],
        out_shape=_STATE_SHAPE,
        compiler_params=pltpu.CompilerParams(
            dimension_semantics=("arbitrary",)),
    )(lp_t, bits, bv, bi)

    return bi2.reshape(_B)
